# baseline (speedup 1.0000x reference)
import functools

import jax
import jax.numpy as jnp
import numpy as _np
from jax import lax
from jax.experimental import pallas as pl
from jax.experimental.pallas import tpu as pltpu
from jax.experimental.pallas import tpu_sc as plsc

NC = 2
NS = 16
NW = NC * NS
LANES = 16

def _interleave_perm(f):
    perm = _np.empty((f,), dtype=_np.int32)
    for g in range(f // 32):
        for l in range(16):
            perm[32 * g + 2 * l] = 32 * g + l
            perm[32 * g + 2 * l + 1] = 32 * g + 16 + l
    return perm


def _ssp(v):
    return jnp.maximum(v, 0.0) + jnp.log1p(jnp.exp(-jnp.abs(v))) - 0.6931471805599453


def _h_body(x_ref, w1t_ref, b1_ref, o_ref):
    o_ref[...] = jnp.dot(x_ref[...], w1t_ref[...],
                         preferred_element_type=jnp.float32) + b1_ref[...]


def _compute_h(x, W1, b1):
    n, d = x.shape
    blk = 1000
    grid = n // blk
    return pl.pallas_call(
        _h_body,
        grid=(grid,),
        in_specs=[
            pl.BlockSpec((blk, d), lambda i: (i, 0)),
            pl.BlockSpec((d, W1.shape[0]), lambda i: (0, 0)),
            pl.BlockSpec((1, W1.shape[0]), lambda i: (0, 0)),
        ],
        out_specs=pl.BlockSpec((blk, W1.shape[0]), lambda i: (i, 0)),
        out_shape=jax.ShapeDtypeStruct((n, W1.shape[0]), jnp.float32),
    )(x, W1.T, b1.reshape(1, -1))


def _wij_body(f_ref, wft_ref, bf_ref, rc_ref, o_ref):
    u = jnp.dot(f_ref[...], wft_ref[...],
                preferred_element_type=jnp.float32) + bf_ref[...]
    o_ref[...] = (_ssp(u) * rc_ref[...]).astype(jnp.bfloat16)


def _compute_wij(f_ij, Wf, bf, rcut):
    p, r = f_ij.shape
    f = Wf.shape[0]
    perm = _interleave_perm(f)
    blk = 4000
    grid = p // blk
    return pl.pallas_call(
        _wij_body,
        grid=(grid,),
        in_specs=[
            pl.BlockSpec((blk, r), lambda i: (i, 0)),
            pl.BlockSpec((r, f), lambda i: (0, 0)),
            pl.BlockSpec((1, f), lambda i: (0, 0)),
            pl.BlockSpec((blk, 1), lambda i: (i, 0)),
        ],
        out_specs=pl.BlockSpec((blk, f), lambda i: (i, 0)),
        out_shape=jax.ShapeDtypeStruct((p, f), jnp.bfloat16),
    )(f_ij, Wf.T[:, perm], bf[perm].reshape(1, -1), rcut.reshape(-1, 1))


def _out_body(p_ref, w2t_ref, b2_ref, o_ref):
    acc = p_ref[0] + p_ref[1]
    o_ref[...] = _ssp(jnp.dot(acc, w2t_ref[...],
                              preferred_element_type=jnp.float32) + b2_ref[...])


def _compute_out(parts, W2, b2):
    _, n, f = parts.shape
    d = W2.shape[0]
    blk = 1000
    grid = n // blk
    return pl.pallas_call(
        _out_body,
        grid=(grid,),
        in_specs=[
            pl.BlockSpec((2, blk, f), lambda i: (0, i, 0)),
            pl.BlockSpec((f, d), lambda i: (0, 0)),
            pl.BlockSpec((1, d), lambda i: (0, 0)),
        ],
        out_specs=pl.BlockSpec((blk, d), lambda i: (i, 0)),
        out_shape=jax.ShapeDtypeStruct((n, d), jnp.float32),
    )(parts, W2.T, b2.reshape(1, -1))


def _make_sc_scatter(n, d, p, chunk):
    nz_tiles = 10
    n_per_tile = n // nz_tiles
    p_per_tile = p // NW
    n_chunks = p_per_tile // chunk
    assert n_chunks % 2 == 1
    mesh = plsc.VectorSubcoreMesh(core_axis_name="c", subcore_axis_name="s")

    @functools.partial(
        pl.kernel,
        out_type=jax.ShapeDtypeStruct((NC, n, d), jnp.float32),
        mesh=mesh,
        scratch_types=[
            pltpu.VMEM((p_per_tile,), jnp.int32),
            pltpu.VMEM((2, chunk), jnp.int32),
            pltpu.VMEM((chunk, d), jnp.float32),
            pltpu.VMEM((chunk, d), jnp.float32),
            pltpu.VMEM((chunk * d // 2,), jnp.int32),
            pltpu.VMEM((chunk * d // 2,), jnp.int32),
            pltpu.VMEM_SHARED((n, d), jnp.float32),
            pltpu.SemaphoreType.DMA,
            pltpu.SemaphoreType.DMA,
            pltpu.SemaphoreType.DMA,
            pltpu.SemaphoreType.DMA,
            pltpu.SemaphoreType.DMA,
            pltpu.SemaphoreType.DMA,
            pltpu.SemaphoreType.DMA,
            pltpu.SemaphoreType.DMA,
        ],
    )
    def sc_kernel(h_hbm, wij_hbm, idxj_hbm, idxi_hbm, zero_hbm, out_hbm,
                  idxj_v, sidx_v, rows0, rows1, wij0, wij1, acc_sh,
                  gsem0, gsem1, wsem0, wsem1, ssem0, ssem1, isem0, isem1):
        rows = (rows0, rows1)
        wij = (wij0, wij1)
        gsem = (gsem0, gsem1)
        wsem = (wsem0, wsem1)
        ssem = (ssem0, ssem1)
        isem = (isem0, isem1)
        c = lax.axis_index("c")
        s = lax.axis_index("s")
        wid = c * NS + s

        row0 = s * n_per_tile

        @pl.when(s < nz_tiles)
        def _():
            pltpu.sync_copy(zero_hbm.at[pl.ds(row0, n_per_tile)],
                            acc_sh.at[pl.ds(row0, n_per_tile)])

        base0 = wid * p_per_tile
        pltpu.sync_copy(idxj_hbm.at[pl.ds(base0, p_per_tile)], idxj_v)
        plsc.subcore_barrier()

        wpc = chunk * d // 2

        def issue(j, b):
            pltpu.async_copy(h_hbm.at[idxj_v.at[pl.ds(j * chunk, chunk)]],
                             rows[b], gsem[b])
            pltpu.async_copy(wij_hbm.at[pl.ds(base0 * (d // 2) + j * wpc, wpc)],
                             wij[b], wsem[b])
            pltpu.async_copy(idxi_hbm.at[pl.ds(base0 + j * chunk, chunk)],
                             sidx_v.at[b], isem[b])

        def step(j, b):
            nb = 1 - b

            @pl.when(j + 1 < n_chunks)
            def _():
                @pl.when(j >= 1)
                def _():
                    pltpu.make_async_copy(
                        rows[nb], acc_sh.at[sidx_v.at[nb]], ssem[nb]).wait()
                issue(j + 1, nb)

            pltpu.make_async_copy(h_hbm.at[idxj_v.at[pl.ds(0, chunk)]],
                                  rows[b], gsem[b]).wait()
            pltpu.make_async_copy(wij_hbm.at[pl.ds(0, wpc)], wij[b],
                                  wsem[b]).wait()

            def mul_body(rr, carry):
                wbase = pl.multiple_of(rr * (d // 2), d // 2)
                for g in range(d // 32):
                    u = lax.bitcast_convert_type(
                        wij[b][pl.ds(wbase + 16 * g, 16)], jnp.uint32)
                    lo = lax.bitcast_convert_type(u << 16, jnp.float32)
                    hi = lax.bitcast_convert_type(u & jnp.uint32(0xFFFF0000),
                                                  jnp.float32)
                    sl0 = pl.ds(32 * g, LANES)
                    sl1 = pl.ds(32 * g + LANES, LANES)
                    rows[b][rr, sl0] = rows[b][rr, sl0] * lo
                    rows[b][rr, sl1] = rows[b][rr, sl1] * hi
                return carry

            lax.fori_loop(0, chunk, mul_body, 0, unroll=2)
            pltpu.make_async_copy(idxi_hbm.at[pl.ds(0, chunk)], sidx_v.at[b],
                                  isem[b]).wait()
            pltpu.async_copy(rows[b], acc_sh.at[sidx_v.at[b]], ssem[b], add=True)

        issue(0, 0)

        def pair_body(t, carry):
            step(2 * t, 0)
            step(2 * t + 1, 1)
            return carry

        lax.fori_loop(0, (n_chunks - 1) // 2, pair_body, 0)
        step(n_chunks - 1, 0)
        pltpu.make_async_copy(rows[1], acc_sh.at[sidx_v.at[1]], ssem[1]).wait()
        pltpu.make_async_copy(rows[0], acc_sh.at[sidx_v.at[0]], ssem[0]).wait()
        plsc.subcore_barrier()

        @pl.when(s < nz_tiles)
        def _():
            pltpu.sync_copy(acc_sh.at[pl.ds(row0, n_per_tile)],
                            out_hbm.at[c, pl.ds(row0, n_per_tile)])

    return sc_kernel


def kernel(x, f_ij, idx_i, idx_j, rcut_ij, W1, b1, Wf, bf, W2, b2):
    n, d = x.shape
    p = f_ij.shape[0]
    chunk = 80
    h = _compute_h(x, W1, b1)
    wij_bf16 = _compute_wij(f_ij, Wf, bf, rcut_ij)
    wij = lax.bitcast_convert_type(wij_bf16.reshape(p, d // 2, 2),
                                   jnp.int32).reshape(p * (d // 2))
    zeros = jnp.zeros((n, d), jnp.float32)
    sc = _make_sc_scatter(n, d, p, chunk=chunk)
    parts = sc(h, wij, idx_j.astype(jnp.int32), idx_i.astype(jnp.int32), zeros)
    return _compute_out(parts, W2, b2)


# speedup vs baseline: 1.9085x; 1.9085x over previous
import functools

import jax
import jax.numpy as jnp
import numpy as _np
from jax import lax
from jax.experimental import pallas as pl
from jax.experimental.pallas import tpu as pltpu
from jax.experimental.pallas import tpu_sc as plsc

NC = 2
NS = 16
NW = NC * NS
LANES = 16

def _interleave_perm(f):
    perm = _np.empty((f,), dtype=_np.int32)
    for g in range(f // 32):
        for l in range(16):
            perm[32 * g + 2 * l] = 32 * g + l
            perm[32 * g + 2 * l + 1] = 32 * g + 16 + l
    return perm


def _ssp(v):
    return jnp.maximum(v, 0.0) + jnp.log1p(jnp.exp(-jnp.abs(v))) - 0.6931471805599453


def _h_body(x_ref, w1t_ref, b1_ref, o_ref):
    o_ref[...] = jnp.dot(x_ref[...], w1t_ref[...],
                         preferred_element_type=jnp.float32) + b1_ref[...]


def _compute_h(x, W1, b1):
    n, d = x.shape
    blk = 1000
    grid = n // blk
    return pl.pallas_call(
        _h_body,
        grid=(grid,),
        in_specs=[
            pl.BlockSpec((blk, d), lambda i: (i, 0)),
            pl.BlockSpec((d, W1.shape[0]), lambda i: (0, 0)),
            pl.BlockSpec((1, W1.shape[0]), lambda i: (0, 0)),
        ],
        out_specs=pl.BlockSpec((blk, W1.shape[0]), lambda i: (i, 0)),
        out_shape=jax.ShapeDtypeStruct((n, W1.shape[0]), jnp.float32),
    )(x, W1.T, b1.reshape(1, -1))


def _wij_body(f_ref, wft_ref, bf_ref, rc_ref, o_ref):
    f = wft_ref.shape[1]
    u = jnp.dot(f_ref[...], wft_ref[...],
                preferred_element_type=jnp.float32) + bf_ref[...]
    v = _ssp(u) * rc_ref[...]
    a = lax.bitcast_convert_type(v[:, :f // 2].astype(jnp.bfloat16),
                                 jnp.uint16).astype(jnp.uint32)
    b = lax.bitcast_convert_type(v[:, f // 2:].astype(jnp.bfloat16),
                                 jnp.uint16).astype(jnp.uint32)
    o_ref[...] = lax.bitcast_convert_type(a | (b << 16), jnp.int32)


def _compute_wij(f_ij, Wf, bf, rcut):
    p, r = f_ij.shape
    f = Wf.shape[0]
    perm = _interleave_perm(f)
    perm2 = _np.concatenate([perm[0::2], perm[1::2]])
    blk = 4000
    grid = p // blk
    return pl.pallas_call(
        _wij_body,
        grid=(grid,),
        in_specs=[
            pl.BlockSpec((blk, r), lambda i: (i, 0)),
            pl.BlockSpec((r, f), lambda i: (0, 0)),
            pl.BlockSpec((1, f), lambda i: (0, 0)),
            pl.BlockSpec((blk, 1), lambda i: (i, 0)),
        ],
        out_specs=pl.BlockSpec((blk, f // 2), lambda i: (i, 0)),
        out_shape=jax.ShapeDtypeStruct((p, f // 2), jnp.int32),
    )(f_ij, Wf.T[:, perm2], bf[perm2].reshape(1, -1), rcut.reshape(-1, 1))


def _out_body(p_ref, w2t_ref, b2_ref, o_ref):
    acc = p_ref[0] + p_ref[1]
    o_ref[...] = _ssp(jnp.dot(acc, w2t_ref[...],
                              preferred_element_type=jnp.float32) + b2_ref[...])


def _compute_out(parts, W2, b2):
    _, n, f = parts.shape
    d = W2.shape[0]
    blk = 1000
    grid = n // blk
    return pl.pallas_call(
        _out_body,
        grid=(grid,),
        in_specs=[
            pl.BlockSpec((2, blk, f), lambda i: (0, i, 0)),
            pl.BlockSpec((f, d), lambda i: (0, 0)),
            pl.BlockSpec((1, d), lambda i: (0, 0)),
        ],
        out_specs=pl.BlockSpec((blk, d), lambda i: (i, 0)),
        out_shape=jax.ShapeDtypeStruct((n, d), jnp.float32),
    )(parts, W2.T, b2.reshape(1, -1))


def _make_sc_scatter(n, d, p, chunk):
    nz_tiles = 10
    n_per_tile = n // nz_tiles
    p_per_tile = p // NW
    n_chunks = p_per_tile // chunk
    assert n_chunks % 2 == 1
    mesh = plsc.VectorSubcoreMesh(core_axis_name="c", subcore_axis_name="s")

    @functools.partial(
        pl.kernel,
        out_type=jax.ShapeDtypeStruct((NC, n, d), jnp.float32),
        mesh=mesh,
        scratch_types=[
            pltpu.VMEM((p_per_tile,), jnp.int32),
            pltpu.VMEM((2, chunk), jnp.int32),
            pltpu.VMEM((chunk, d), jnp.float32),
            pltpu.VMEM((chunk, d), jnp.float32),
            pltpu.VMEM((chunk * d // 2,), jnp.int32),
            pltpu.VMEM((chunk * d // 2,), jnp.int32),
            pltpu.VMEM_SHARED((n, d), jnp.float32),
            pltpu.SemaphoreType.DMA,
            pltpu.SemaphoreType.DMA,
            pltpu.SemaphoreType.DMA,
            pltpu.SemaphoreType.DMA,
            pltpu.SemaphoreType.DMA,
            pltpu.SemaphoreType.DMA,
            pltpu.SemaphoreType.DMA,
            pltpu.SemaphoreType.DMA,
        ],
    )
    def sc_kernel(h_hbm, wij_hbm, idxj_hbm, idxi_hbm, zero_hbm, out_hbm,
                  idxj_v, sidx_v, rows0, rows1, wij0, wij1, acc_sh,
                  gsem0, gsem1, wsem0, wsem1, ssem0, ssem1, isem0, isem1):
        rows = (rows0, rows1)
        wij = (wij0, wij1)
        gsem = (gsem0, gsem1)
        wsem = (wsem0, wsem1)
        ssem = (ssem0, ssem1)
        isem = (isem0, isem1)
        c = lax.axis_index("c")
        s = lax.axis_index("s")
        wid = c * NS + s

        row0 = s * n_per_tile

        @pl.when(s < nz_tiles)
        def _():
            pltpu.sync_copy(zero_hbm.at[pl.ds(row0, n_per_tile)],
                            acc_sh.at[pl.ds(row0, n_per_tile)])

        base0 = wid * p_per_tile
        pltpu.sync_copy(idxj_hbm.at[pl.ds(base0, p_per_tile)], idxj_v)
        plsc.subcore_barrier()

        wpc = chunk * d // 2

        def issue(j, b):
            pltpu.async_copy(h_hbm.at[idxj_v.at[pl.ds(j * chunk, chunk)]],
                             rows[b], gsem[b])
            pltpu.async_copy(wij_hbm.at[pl.ds(base0 * (d // 2) + j * wpc, wpc)],
                             wij[b], wsem[b])
            pltpu.async_copy(idxi_hbm.at[pl.ds(base0 + j * chunk, chunk)],
                             sidx_v.at[b], isem[b])

        def step(j, b):
            nb = 1 - b

            @pl.when(j + 1 < n_chunks)
            def _():
                @pl.when(j >= 1)
                def _():
                    pltpu.make_async_copy(
                        rows[nb], acc_sh.at[sidx_v.at[nb]], ssem[nb]).wait()
                issue(j + 1, nb)

            pltpu.make_async_copy(h_hbm.at[idxj_v.at[pl.ds(0, chunk)]],
                                  rows[b], gsem[b]).wait()
            pltpu.make_async_copy(wij_hbm.at[pl.ds(0, wpc)], wij[b],
                                  wsem[b]).wait()

            def mul_body(rr, carry):
                wbase = pl.multiple_of(rr * (d // 2), d // 2)
                for g in range(d // 32):
                    u = lax.bitcast_convert_type(
                        wij[b][pl.ds(wbase + 16 * g, 16)], jnp.uint32)
                    lo = lax.bitcast_convert_type(u << 16, jnp.float32)
                    hi = lax.bitcast_convert_type(u & jnp.uint32(0xFFFF0000),
                                                  jnp.float32)
                    sl0 = pl.ds(32 * g, LANES)
                    sl1 = pl.ds(32 * g + LANES, LANES)
                    rows[b][rr, sl0] = rows[b][rr, sl0] * lo
                    rows[b][rr, sl1] = rows[b][rr, sl1] * hi
                return carry

            lax.fori_loop(0, chunk, mul_body, 0, unroll=2)
            pltpu.make_async_copy(idxi_hbm.at[pl.ds(0, chunk)], sidx_v.at[b],
                                  isem[b]).wait()
            pltpu.async_copy(rows[b], acc_sh.at[sidx_v.at[b]], ssem[b], add=True)

        issue(0, 0)

        def pair_body(t, carry):
            step(2 * t, 0)
            step(2 * t + 1, 1)
            return carry

        lax.fori_loop(0, (n_chunks - 1) // 2, pair_body, 0)
        step(n_chunks - 1, 0)
        pltpu.make_async_copy(rows[1], acc_sh.at[sidx_v.at[1]], ssem[1]).wait()
        pltpu.make_async_copy(rows[0], acc_sh.at[sidx_v.at[0]], ssem[0]).wait()
        plsc.subcore_barrier()

        @pl.when(s < nz_tiles)
        def _():
            pltpu.sync_copy(acc_sh.at[pl.ds(row0, n_per_tile)],
                            out_hbm.at[c, pl.ds(row0, n_per_tile)])

    return sc_kernel


def kernel(x, f_ij, idx_i, idx_j, rcut_ij, W1, b1, Wf, bf, W2, b2):
    n, d = x.shape
    p = f_ij.shape[0]
    chunk = 80
    h = _compute_h(x, W1, b1)
    wij = _compute_wij(f_ij, Wf, bf, rcut_ij).reshape(p * (d // 2))
    zeros = jnp.zeros((n, d), jnp.float32)
    sc = _make_sc_scatter(n, d, p, chunk=chunk)
    parts = sc(h, wij, idx_j.astype(jnp.int32), idx_i.astype(jnp.int32), zeros)
    return _compute_out(parts, W2, b2)
